# baseline (speedup 1.0000x reference)
import functools

import jax
import jax.numpy as jnp
from jax import lax
from jax.experimental import pallas as pl
from jax.experimental.pallas import tpu as pltpu
from jax.experimental.pallas import tpu_sc as plsc

B = 4096
L = 50
V = 100000
D = 128
A = 14
SPA = 30
S = A * SPA
LS = 3
ANPHA = 0.5

NW = 32
NB = B // NW
CH = 4
NCHUNK = NB // CH
SEEDS_PER_TILE = 16
S_PAD = NW * SEEDS_PER_TILE
NEG = -1e9


def _tc_pvec(emb_t, emb_s, att_sent):
    VB = 10000

    def body(et_ref, es_ref, att_ref, pt_ref, ps_ref):
        a = att_ref[...]
        pt_ref[...] = jnp.sum(et_ref[...] * a, axis=1, keepdims=True)
        ps_ref[...] = jnp.sum(es_ref[...] * a, axis=1, keepdims=True)

    pt, ps = pl.pallas_call(
        body,
        grid=(V // VB,),
        in_specs=[
            pl.BlockSpec((VB, D), lambda i: (i, 0)),
            pl.BlockSpec((VB, D), lambda i: (i, 0)),
            pl.BlockSpec((1, D), lambda i: (0, 0)),
        ],
        out_specs=[
            pl.BlockSpec((VB, 1), lambda i: (i, 0)),
            pl.BlockSpec((VB, 1), lambda i: (i, 0)),
        ],
        out_shape=[jax.ShapeDtypeStruct((V, 1), jnp.float32)] * 2,
    )(emb_t, emb_s, att_sent.reshape(1, D))
    return pt.reshape(V), ps.reshape(V)


def _sc_embedding_stage(sents, mask, seeds_flat, att_sent, emb_t, emb_s):
    mask64 = jnp.pad(mask, ((0, 0), (0, 64 - L)))
    sents = jnp.where(mask > 0.0, sents, 0)
    p_t, p_s = _tc_pvec(emb_t, emb_s, att_sent)
    mesh = plsc.VectorSubcoreMesh(core_axis_name="c", subcore_axis_name="s")

    @functools.partial(
        pl.kernel,
        out_type=(
            jax.ShapeDtypeStruct((B, D), jnp.float32),
            jax.ShapeDtypeStruct((B, D), jnp.float32),
            jax.ShapeDtypeStruct((S_PAD, D), jnp.float32),
            jax.ShapeDtypeStruct((S_PAD, D), jnp.float32),
        ),
        mesh=mesh,
        compiler_params=pltpu.CompilerParams(needs_layout_passes=False,
                                             use_tc_tiling_on_sc=False),
        scratch_types=(
            [pltpu.VMEM((CH, L), jnp.int32)] * 2
            + [pltpu.VMEM((CH, 64), jnp.float32)] * 2
            + [pltpu.VMEM((CH, L, D), jnp.float32)] * 2
            + [pltpu.VMEM((CH, L, D), jnp.float32)] * 2
            + [pltpu.VMEM((CH, L), jnp.float32)] * 2
            + [pltpu.VMEM((CH, L), jnp.float32)] * 2
            + [pltpu.VMEM((CH, D), jnp.float32)] * 2
            + [pltpu.VMEM((CH, D), jnp.float32)] * 2
            + [
                pltpu.VMEM((LS * SEEDS_PER_TILE,), jnp.int32),
                pltpu.VMEM((LS * SEEDS_PER_TILE, D), jnp.float32),
                pltpu.VMEM((SEEDS_PER_TILE, D), jnp.float32),
            ]
            + [pltpu.SemaphoreType.DMA] * 6
        ),
    )
    def sc_kernel(sents_hbm, mask_hbm, seeds_hbm, pt_hbm, ps_hbm,
                  embt_hbm, embs_hbm,
                  snt_t_hbm, snt_s_hbm, sd_t_hbm, sd_s_hbm,
                  idx0, idx1, msk0, msk1, rt0, rt1, rs0, rs1,
                  st0, st1, ss0, ss1, ot0, ot1, os0, os1,
                  sidx_v, srows_v, ssd_v,
                  sem_a0, sem_a1, sem_b0, sem_b1, sem_p0, sem_p1):
        idxs, msks = [idx0, idx1], [msk0, msk1]
        rows_t, rows_s = [rt0, rt1], [rs0, rs1]
        sc_t, sc_s = [st0, st1], [ss0, ss1]
        out_t, out_s = [ot0, ot1], [os0, os1]
        sems_a, sems_b = [sem_a0, sem_a1], [sem_b0, sem_b1]
        sems_p = [sem_p0, sem_p1]
        wid = lax.axis_index("s") * 2 + lax.axis_index("c")

        pltpu.sync_copy(
            seeds_hbm.at[pl.ds(wid * (LS * SEEDS_PER_TILE), LS * SEEDS_PER_TILE)],
            sidx_v)
        for table_hbm, out_hbm in ((embt_hbm, sd_t_hbm), (embs_hbm, sd_s_hbm)):
            pltpu.async_copy(table_hbm.at[sidx_v], srows_v, sem_a0).wait()
            for j in range(SEEDS_PER_TILE):
                for chk in range(D // 16):
                    sl = pl.ds(chk * 16, 16)
                    acc = (srows_v[3 * j, sl] + srows_v[3 * j + 1, sl]
                           + srows_v[3 * j + 2, sl]) * (1.0 / 3.0)
                    ssd_v[j, sl] = acc
            pltpu.sync_copy(
                ssd_v, out_hbm.at[pl.ds(wid * SEEDS_PER_TILE, SEEDS_PER_TILE)])

        NG = 4

        lane16 = lax.iota(jnp.int32, 16)
        tail_idx = jnp.minimum(lane16 + 48, L - 1)

        def softmax_c(sref, mref, c):
            cvec = jnp.full((16,), c, jnp.int32)
            raw = [sref[c, pl.ds(g * 16, 16)] for g in range(3)]
            raw.append(plsc.load_gather(sref, [cvec, tail_idx]))
            svs = [jnp.where(mref[c, pl.ds(g * 16, 16)] > 0.0, raw[g], NEG)
                   for g in range(NG)]
            m = jnp.max(jnp.maximum(jnp.maximum(svs[0], svs[1]),
                                    jnp.maximum(svs[2], svs[3])))
            es = [jnp.exp(svs[g] - m) for g in range(NG)]
            z = jnp.sum(es[0] + es[1] + es[2] + es[3])
            for g in range(3):
                sref[c, pl.ds(g * 16, 16)] = es[g] / z
            plsc.store_scatter(sref, [cvec, tail_idx], es[3] / z,
                               mask=lane16 < 2)

        def bag(rows_ref, sref, c, out_ref):
            cvec = jnp.full((16,), c, jnp.int32)

            @plsc.parallel_loop(
                0, L, unroll=2,
                carry=tuple(jnp.zeros((16,), jnp.float32)
                            for _ in range(D // 16)))
            def acc(l, carry):
                wv = plsc.load_gather(sref, [cvec, jnp.full((16,), l, jnp.int32)])
                return tuple(carry[chk] + wv * rows_ref[c, l, pl.ds(chk * 16, 16)]
                             for chk in range(D // 16))

            for chk in range(D // 16):
                out_ref[c, pl.ds(chk * 16, 16)] = acc[chk]

        base = wid * NB

        def handles(k, s):
            b0 = base + k * CH
            hp = [pltpu.make_async_copy(pt_hbm.at[idxs[s].at[c]],
                                        sc_t[s].at[c], sems_p[s])
                  for c in range(CH)]
            hp += [pltpu.make_async_copy(ps_hbm.at[idxs[s].at[c]],
                                         sc_s[s].at[c], sems_p[s])
                   for c in range(CH)]
            ht = [pltpu.make_async_copy(embt_hbm.at[idxs[s].at[c]],
                                        rows_t[s].at[c], sems_a[s])
                  for c in range(CH)]
            hs = [pltpu.make_async_copy(embs_hbm.at[idxs[s].at[c]],
                                        rows_s[s].at[c], sems_b[s])
                  for c in range(CH)]
            del b0
            return hp, ht, hs

        def issue(k, s):
            b0 = base + k * CH
            pltpu.sync_copy(sents_hbm.at[pl.ds(b0, CH)], idxs[s])
            pltpu.sync_copy(mask_hbm.at[pl.ds(b0, CH)], msks[s])
            hp, ht, hs = handles(k, s)
            for h in hp + ht + hs:
                h.start()

        def consume(k, s):
            b0 = base + k * CH
            hp, ht, hs = handles(k, s)
            for h in hp:
                h.wait()
            for c in range(CH):
                softmax_c(sc_t[s], msks[s], c)
                softmax_c(sc_s[s], msks[s], c)
            for h in ht:
                h.wait()
            for c in range(CH):
                bag(rows_t[s], sc_t[s], c, out_t[s])
            for h in hs:
                h.wait()
            for c in range(CH):
                bag(rows_s[s], sc_s[s], c, out_s[s])
            pltpu.sync_copy(out_t[s], snt_t_hbm.at[pl.ds(b0, CH)])
            pltpu.sync_copy(out_s[s], snt_s_hbm.at[pl.ds(b0, CH)])

        issue(0, 0)

        def pair_body(i, _):
            k0 = i * 2
            issue(k0 + 1, 1)
            consume(k0, 0)
            issue(k0 + 2, 0)
            consume(k0 + 1, 1)
            return 0

        lax.fori_loop(0, NCHUNK // 2 - 1, pair_body, 0)
        issue(NCHUNK - 1, 1)
        consume(NCHUNK - 2, 0)
        consume(NCHUNK - 1, 1)

    return sc_kernel(sents, mask64, seeds_flat, p_t, p_s, emb_t, emb_s)


def _tc_tail(snt_t, snt_s, sd_t, sd_s):
    BSZ = 512
    dn = (((1,), (1,)), ((), ()))

    def tc_body(snt_t_ref, snt_s_ref, sd_t_ref, sd_s_ref, out_ref):
        pro_t = lax.dot_general(snt_t_ref[...], sd_t_ref[...], dn,
                                preferred_element_type=jnp.float32)
        pro_s = lax.dot_general(snt_s_ref[...], sd_s_ref[...], dn,
                                preferred_element_type=jnp.float32)
        gt = jnp.concatenate(
            [jnp.max(pro_t[:, a * SPA:(a + 1) * SPA], axis=1, keepdims=True)
             for a in range(A)], axis=1)
        gs = jnp.concatenate(
            [jnp.max(pro_s[:, a * SPA:(a + 1) * SPA], axis=1, keepdims=True)
             for a in range(A)], axis=1)
        mt = jnp.max(gt, axis=1, keepdims=True)
        et = jnp.exp(gt - mt)
        pt = et / jnp.sum(et, axis=1, keepdims=True)
        reli = jnp.max(pt, axis=1, keepdims=True) - 1.0 / A
        dd = pt - gs
        out_ref[0, 0, 0] = jnp.sum((1.0 + ANPHA * jnp.abs(reli)) * dd * dd)

    return pl.pallas_call(
        tc_body,
        grid=(B // BSZ,),
        in_specs=[
            pl.BlockSpec((BSZ, D), lambda i: (i, 0)),
            pl.BlockSpec((BSZ, D), lambda i: (i, 0)),
            pl.BlockSpec((S_PAD, D), lambda i: (0, 0)),
            pl.BlockSpec((S_PAD, D), lambda i: (0, 0)),
        ],
        out_specs=pl.BlockSpec((1, 1, 1), lambda i: (i, 0, 0),
                               memory_space=pltpu.SMEM),
        out_shape=jax.ShapeDtypeStruct((B // BSZ, 1, 1), jnp.float32),
    )(snt_t, snt_s, sd_t, sd_s)


def kernel(sents, seeds, num_clusters, num_arr, mask, flag,
           emb_teacher, emb_student, att_sent):
    del num_clusters, num_arr
    seeds_flat = jnp.pad(seeds, ((0, S_PAD - S), (0, 0))).reshape(-1)
    snt_t, snt_s, sd_t, sd_s = _sc_embedding_stage(
        sents, mask, seeds_flat, att_sent, emb_teacher, emb_student)
    partials = _tc_tail(snt_t, snt_s, sd_t, sd_s)
    return jnp.sum(partials) / B * flag


# speedup vs baseline: 12.4870x; 12.4870x over previous
import functools

import jax
import jax.numpy as jnp
from jax import lax
from jax.experimental import pallas as pl
from jax.experimental.pallas import tpu as pltpu
from jax.experimental.pallas import tpu_sc as plsc

B = 4096
L = 50
V = 100000
D = 128
A = 14
SPA = 30
S = A * SPA
LS = 3
ANPHA = 0.5

NW = 32
NB = B // NW
CH = 4
NCHUNK = NB // CH
SEEDS_PER_TILE = 16
S_PAD = NW * SEEDS_PER_TILE
NEG = -1e9


def _tc_pvec(emb_t, emb_s, att_sent):
    VB = 10000

    def body(et_ref, es_ref, att_ref, pt_ref, ps_ref):
        a = att_ref[...]
        pt_ref[...] = jnp.sum(et_ref[...] * a, axis=1, keepdims=True)
        ps_ref[...] = jnp.sum(es_ref[...] * a, axis=1, keepdims=True)

    pt, ps = pl.pallas_call(
        body,
        grid=(V // VB,),
        in_specs=[
            pl.BlockSpec((VB, D), lambda i: (i, 0)),
            pl.BlockSpec((VB, D), lambda i: (i, 0)),
            pl.BlockSpec((1, D), lambda i: (0, 0)),
        ],
        out_specs=[
            pl.BlockSpec((VB, 1), lambda i: (i, 0)),
            pl.BlockSpec((VB, 1), lambda i: (i, 0)),
        ],
        out_shape=[jax.ShapeDtypeStruct((V, 1), jnp.float32)] * 2,
    )(emb_t, emb_s, att_sent.reshape(1, D))
    return pt.reshape(V), ps.reshape(V)


def _sc_embedding_stage(sents, mask, seeds_flat, att_sent, emb_t, emb_s):
    mask64 = jnp.pad(mask, ((0, 0), (0, 64 - L)))
    p_t, p_s = _tc_pvec(emb_t, emb_s, att_sent)
    mesh = plsc.VectorSubcoreMesh(core_axis_name="c", subcore_axis_name="s")

    @functools.partial(
        pl.kernel,
        out_type=(
            jax.ShapeDtypeStruct((B, D), jnp.float32),
            jax.ShapeDtypeStruct((B, D), jnp.float32),
            jax.ShapeDtypeStruct((S_PAD, D), jnp.float32),
            jax.ShapeDtypeStruct((S_PAD, D), jnp.float32),
        ),
        mesh=mesh,
        compiler_params=pltpu.CompilerParams(needs_layout_passes=False,
                                             use_tc_tiling_on_sc=False),
        scratch_types=(
            [pltpu.VMEM((CH, L), jnp.int32)] * 2
            + [pltpu.VMEM((CH, 64), jnp.float32)] * 2
            + [pltpu.VMEM((CH, L, D), jnp.float32)] * 2
            + [pltpu.VMEM((CH, L, D), jnp.float32)] * 2
            + [pltpu.VMEM((CH, L), jnp.float32)] * 2
            + [pltpu.VMEM((CH, L), jnp.float32)] * 2
            + [pltpu.VMEM((CH, D), jnp.float32)] * 2
            + [pltpu.VMEM((CH, D), jnp.float32)] * 2
            + [
                pltpu.VMEM((LS * SEEDS_PER_TILE,), jnp.int32),
                pltpu.VMEM((LS * SEEDS_PER_TILE, D), jnp.float32),
                pltpu.VMEM((SEEDS_PER_TILE, D), jnp.float32),
            ]
            + [pltpu.SemaphoreType.DMA] * 6
        ),
    )
    def sc_kernel(sents_hbm, mask_hbm, seeds_hbm, pt_hbm, ps_hbm,
                  embt_hbm, embs_hbm,
                  snt_t_hbm, snt_s_hbm, sd_t_hbm, sd_s_hbm,
                  idx0, idx1, msk0, msk1, rt0, rt1, rs0, rs1,
                  st0, st1, ss0, ss1, ot0, ot1, os0, os1,
                  sidx_v, srows_v, ssd_v,
                  sem_a0, sem_a1, sem_b0, sem_b1, sem_p0, sem_p1):
        idxs, msks = [idx0, idx1], [msk0, msk1]
        rows_t, rows_s = [rt0, rt1], [rs0, rs1]
        sc_t, sc_s = [st0, st1], [ss0, ss1]
        out_t, out_s = [ot0, ot1], [os0, os1]
        sems_a, sems_b = [sem_a0, sem_a1], [sem_b0, sem_b1]
        sems_p = [sem_p0, sem_p1]
        wid = lax.axis_index("s") * 2 + lax.axis_index("c")

        pltpu.sync_copy(
            seeds_hbm.at[pl.ds(wid * (LS * SEEDS_PER_TILE), LS * SEEDS_PER_TILE)],
            sidx_v)
        for table_hbm, out_hbm in ((embt_hbm, sd_t_hbm), (embs_hbm, sd_s_hbm)):
            pltpu.async_copy(table_hbm.at[sidx_v], srows_v, sem_a0).wait()
            for j in range(SEEDS_PER_TILE):
                for chk in range(D // 16):
                    sl = pl.ds(chk * 16, 16)
                    acc = (srows_v[3 * j, sl] + srows_v[3 * j + 1, sl]
                           + srows_v[3 * j + 2, sl]) * (1.0 / 3.0)
                    ssd_v[j, sl] = acc
            pltpu.sync_copy(
                ssd_v, out_hbm.at[pl.ds(wid * SEEDS_PER_TILE, SEEDS_PER_TILE)])

        NG = 4

        lane16 = lax.iota(jnp.int32, 16)
        tail_idx = jnp.minimum(lane16 + 48, L - 1)

        def softmax_c(sref, mref, c):
            cvec = jnp.full((16,), c, jnp.int32)
            raw = [sref[c, pl.ds(g * 16, 16)] for g in range(3)]
            raw.append(plsc.load_gather(sref, [cvec, tail_idx]))
            svs = [jnp.where(mref[c, pl.ds(g * 16, 16)] > 0.0, raw[g], NEG)
                   for g in range(NG)]
            m = jnp.max(jnp.maximum(jnp.maximum(svs[0], svs[1]),
                                    jnp.maximum(svs[2], svs[3])))
            es = [jnp.exp(svs[g] - m) for g in range(NG)]
            z = jnp.sum(es[0] + es[1] + es[2] + es[3])
            for g in range(3):
                sref[c, pl.ds(g * 16, 16)] = es[g] / z
            plsc.store_scatter(sref, [cvec, tail_idx], es[3] / z,
                               mask=lane16 < 2)

        def bag(rows_ref, sref, c, lenc, out_ref):
            cvec = jnp.full((16,), c, jnp.int32)

            @plsc.parallel_loop(
                0, lenc, unroll=2,
                carry=tuple(jnp.zeros((16,), jnp.float32)
                            for _ in range(D // 16)))
            def acc(l, carry):
                wv = plsc.load_gather(sref, [cvec, jnp.full((16,), l, jnp.int32)])
                return tuple(carry[chk] + wv * rows_ref[c, l, pl.ds(chk * 16, 16)]
                             for chk in range(D // 16))

            for chk in range(D // 16):
                out_ref[c, pl.ds(chk * 16, 16)] = acc[chk]

        base = wid * NB

        def handles(k, s):
            b0 = base + k * CH
            hp = [pltpu.make_async_copy(pt_hbm.at[idxs[s].at[c]],
                                        sc_t[s].at[c], sems_p[s])
                  for c in range(CH)]
            hp += [pltpu.make_async_copy(ps_hbm.at[idxs[s].at[c]],
                                         sc_s[s].at[c], sems_p[s])
                   for c in range(CH)]
            ht = [pltpu.make_async_copy(embt_hbm.at[idxs[s].at[c]],
                                        rows_t[s].at[c], sems_a[s])
                  for c in range(CH)]
            hs = [pltpu.make_async_copy(embs_hbm.at[idxs[s].at[c]],
                                        rows_s[s].at[c], sems_b[s])
                  for c in range(CH)]
            del b0
            return hp, ht, hs

        def issue(k, s):
            b0 = base + k * CH
            pltpu.sync_copy(sents_hbm.at[pl.ds(b0, CH)], idxs[s])
            pltpu.sync_copy(mask_hbm.at[pl.ds(b0, CH)], msks[s])
            hp, ht, hs = handles(k, s)
            for h in hp + ht + hs:
                h.start()

        def consume(k, s):
            b0 = base + k * CH
            hp, ht, hs = handles(k, s)
            for h in hp:
                h.wait()
            for c in range(CH):
                softmax_c(sc_t[s], msks[s], c)
                softmax_c(sc_s[s], msks[s], c)
            lens = []
            for c in range(CH):
                mrow = (msks[s][c, pl.ds(0, 16)] + msks[s][c, pl.ds(16, 16)]
                        + msks[s][c, pl.ds(32, 16)] + msks[s][c, pl.ds(48, 16)])
                lens.append(jnp.sum(mrow).astype(jnp.int32))
            for h in ht:
                h.wait()
            for c in range(CH):
                bag(rows_t[s], sc_t[s], c, lens[c], out_t[s])
            for h in hs:
                h.wait()
            for c in range(CH):
                bag(rows_s[s], sc_s[s], c, lens[c], out_s[s])
            pltpu.sync_copy(out_t[s], snt_t_hbm.at[pl.ds(b0, CH)])
            pltpu.sync_copy(out_s[s], snt_s_hbm.at[pl.ds(b0, CH)])

        issue(0, 0)

        def pair_body(i, _):
            k0 = i * 2
            issue(k0 + 1, 1)
            consume(k0, 0)
            issue(k0 + 2, 0)
            consume(k0 + 1, 1)
            return 0

        lax.fori_loop(0, NCHUNK // 2 - 1, pair_body, 0)
        issue(NCHUNK - 1, 1)
        consume(NCHUNK - 2, 0)
        consume(NCHUNK - 1, 1)

    return sc_kernel(sents, mask64, seeds_flat, p_t, p_s, emb_t, emb_s)


def _tc_tail(snt_t, snt_s, sd_t, sd_s):
    BSZ = 512
    dn = (((1,), (1,)), ((), ()))

    def tc_body(snt_t_ref, snt_s_ref, sd_t_ref, sd_s_ref, out_ref):
        pro_t = lax.dot_general(snt_t_ref[...], sd_t_ref[...], dn,
                                preferred_element_type=jnp.float32)
        pro_s = lax.dot_general(snt_s_ref[...], sd_s_ref[...], dn,
                                preferred_element_type=jnp.float32)
        gt = jnp.concatenate(
            [jnp.max(pro_t[:, a * SPA:(a + 1) * SPA], axis=1, keepdims=True)
             for a in range(A)], axis=1)
        gs = jnp.concatenate(
            [jnp.max(pro_s[:, a * SPA:(a + 1) * SPA], axis=1, keepdims=True)
             for a in range(A)], axis=1)
        mt = jnp.max(gt, axis=1, keepdims=True)
        et = jnp.exp(gt - mt)
        pt = et / jnp.sum(et, axis=1, keepdims=True)
        reli = jnp.max(pt, axis=1, keepdims=True) - 1.0 / A
        dd = pt - gs
        out_ref[0, 0, 0] = jnp.sum((1.0 + ANPHA * jnp.abs(reli)) * dd * dd)

    return pl.pallas_call(
        tc_body,
        grid=(B // BSZ,),
        in_specs=[
            pl.BlockSpec((BSZ, D), lambda i: (i, 0)),
            pl.BlockSpec((BSZ, D), lambda i: (i, 0)),
            pl.BlockSpec((S_PAD, D), lambda i: (0, 0)),
            pl.BlockSpec((S_PAD, D), lambda i: (0, 0)),
        ],
        out_specs=pl.BlockSpec((1, 1, 1), lambda i: (i, 0, 0),
                               memory_space=pltpu.SMEM),
        out_shape=jax.ShapeDtypeStruct((B // BSZ, 1, 1), jnp.float32),
    )(snt_t, snt_s, sd_t, sd_s)


def kernel(sents, seeds, num_clusters, num_arr, mask, flag,
           emb_teacher, emb_student, att_sent):
    del num_clusters, num_arr
    seeds_flat = jnp.pad(seeds, ((0, S_PAD - S), (0, 0))).reshape(-1)
    snt_t, snt_s, sd_t, sd_s = _sc_embedding_stage(
        sents, mask, seeds_flat, att_sent, emb_teacher, emb_student)
    partials = _tc_tail(snt_t, snt_s, sd_t, sd_s)
    return jnp.sum(partials) / B * flag


# speedup vs baseline: 15.2228x; 1.2191x over previous
import functools

import jax
import jax.numpy as jnp
from jax import lax
from jax.experimental import pallas as pl
from jax.experimental.pallas import tpu as pltpu
from jax.experimental.pallas import tpu_sc as plsc

B = 4096
L = 50
V = 100000
D = 128
A = 14
SPA = 30
S = A * SPA
LS = 3
ANPHA = 0.5

NW = 32
NB = B // NW
CH = 4
NCHUNK = NB // CH
SEEDS_PER_TILE = 16
S_PAD = NW * SEEDS_PER_TILE
NEG = -1e9


def _sc_embedding_stage(sents, mask, seeds_flat, att_sent, emb_t, emb_s):
    mask64 = jnp.pad(mask, ((0, 0), (0, 64 - L)))
    mesh = plsc.VectorSubcoreMesh(core_axis_name="c", subcore_axis_name="s")

    @functools.partial(
        pl.kernel,
        out_type=(
            jax.ShapeDtypeStruct((B, D), jnp.float32),
            jax.ShapeDtypeStruct((B, D), jnp.float32),
            jax.ShapeDtypeStruct((S_PAD, D), jnp.float32),
            jax.ShapeDtypeStruct((S_PAD, D), jnp.float32),
        ),
        mesh=mesh,
        compiler_params=pltpu.CompilerParams(needs_layout_passes=False,
                                             use_tc_tiling_on_sc=False),
        scratch_types=(
            [pltpu.VMEM((CH, L), jnp.int32)] * 2
            + [pltpu.VMEM((CH, 64), jnp.float32)] * 2
            + [pltpu.VMEM((CH, L, D), jnp.float32)] * 2
            + [pltpu.VMEM((CH, L, D), jnp.float32)] * 2
            + [pltpu.VMEM((CH, L), jnp.float32)] * 2
            + [pltpu.VMEM((CH, L), jnp.float32)] * 2
            + [pltpu.VMEM((CH, D), jnp.float32)] * 2
            + [pltpu.VMEM((CH, D), jnp.float32)] * 2
            + [
                pltpu.VMEM((D,), jnp.float32),
                pltpu.VMEM((LS * SEEDS_PER_TILE,), jnp.int32),
                pltpu.VMEM((LS * SEEDS_PER_TILE, D), jnp.float32),
                pltpu.VMEM((SEEDS_PER_TILE, D), jnp.float32),
            ]
            + [pltpu.SemaphoreType.DMA] * 4
        ),
    )
    def sc_kernel(sents_hbm, mask_hbm, seeds_hbm, att_hbm,
                  embt_hbm, embs_hbm,
                  snt_t_hbm, snt_s_hbm, sd_t_hbm, sd_s_hbm,
                  idx0, idx1, msk0, msk1, rt0, rt1, rs0, rs1,
                  st0, st1, ss0, ss1, ot0, ot1, os0, os1,
                  att_v, sidx_v, srows_v, ssd_v,
                  sem_a0, sem_a1, sem_b0, sem_b1):
        idxs, msks = [idx0, idx1], [msk0, msk1]
        rows_t, rows_s = [rt0, rt1], [rs0, rs1]
        sc_t, sc_s = [st0, st1], [ss0, ss1]
        out_t, out_s = [ot0, ot1], [os0, os1]
        sems_a, sems_b = [sem_a0, sem_a1], [sem_b0, sem_b1]
        wid = lax.axis_index("s") * 2 + lax.axis_index("c")
        pltpu.sync_copy(att_hbm, att_v)

        pltpu.sync_copy(
            seeds_hbm.at[pl.ds(wid * (LS * SEEDS_PER_TILE), LS * SEEDS_PER_TILE)],
            sidx_v)
        for table_hbm, out_hbm in ((embt_hbm, sd_t_hbm), (embs_hbm, sd_s_hbm)):
            pltpu.async_copy(table_hbm.at[sidx_v], srows_v, sem_a0).wait()

            def seed_body(j, _):
                for chk in range(D // 16):
                    sl = pl.ds(chk * 16, 16)
                    acc = (srows_v[3 * j, sl] + srows_v[3 * j + 1, sl]
                           + srows_v[3 * j + 2, sl]) * (1.0 / 3.0)
                    ssd_v[j, sl] = acc
                return 0

            lax.fori_loop(0, SEEDS_PER_TILE, seed_body, 0)
            pltpu.sync_copy(
                ssd_v, out_hbm.at[pl.ds(wid * SEEDS_PER_TILE, SEEDS_PER_TILE)])

        NG = 4

        lane16 = lax.iota(jnp.int32, 16)
        tail_idx = jnp.minimum(lane16 + 48, L - 1)

        negv = jnp.full((16,), NEG, jnp.float32)

        def scores_c(rows_ref, sref, c, lenc):
            cvec = jnp.full((16,), c, jnp.int32)
            for g in range(3):
                sref[c, pl.ds(g * 16, 16)] = negv
            plsc.store_scatter(sref, [cvec, tail_idx], negv, mask=lane16 < 2)

            @plsc.parallel_loop(0, lenc, unroll=2)
            def _(l):
                acc = None
                for chk in range(D // 16):
                    sl = pl.ds(chk * 16, 16)
                    t = rows_ref[c, l, sl] * att_v[sl]
                    acc = t if acc is None else acc + t
                s = jnp.sum(acc)
                plsc.store_scatter(sref, [cvec, jnp.full((16,), l, jnp.int32)],
                                   jnp.full((16,), s, jnp.float32),
                                   mask=lane16 < 1)

        def softmax_c(sref, c):
            cvec = jnp.full((16,), c, jnp.int32)
            svs = [sref[c, pl.ds(g * 16, 16)] for g in range(3)]
            svs.append(jnp.where(lane16 < 2,
                                 plsc.load_gather(sref, [cvec, tail_idx]), NEG))
            m = jnp.max(jnp.maximum(jnp.maximum(svs[0], svs[1]),
                                    jnp.maximum(svs[2], svs[3])))
            es = [jnp.exp(svs[g] - m) for g in range(NG)]
            z = jnp.sum(es[0] + es[1] + es[2] + es[3])
            for g in range(3):
                sref[c, pl.ds(g * 16, 16)] = es[g] / z
            plsc.store_scatter(sref, [cvec, tail_idx], es[3] / z,
                               mask=lane16 < 2)

        def bag(rows_ref, sref, c, lenc, out_ref):
            cvec = jnp.full((16,), c, jnp.int32)

            @plsc.parallel_loop(
                0, lenc, unroll=2,
                carry=tuple(jnp.zeros((16,), jnp.float32)
                            for _ in range(D // 16)))
            def acc(l, carry):
                wv = plsc.load_gather(sref, [cvec, jnp.full((16,), l, jnp.int32)])
                return tuple(carry[chk] + wv * rows_ref[c, l, pl.ds(chk * 16, 16)]
                             for chk in range(D // 16))

            for chk in range(D // 16):
                out_ref[c, pl.ds(chk * 16, 16)] = acc[chk]

        base = wid * NB

        def handles(k, s):
            ht = [pltpu.make_async_copy(embt_hbm.at[idxs[s].at[c]],
                                        rows_t[s].at[c], sems_a[s])
                  for c in range(CH)]
            hs = [pltpu.make_async_copy(embs_hbm.at[idxs[s].at[c]],
                                        rows_s[s].at[c], sems_b[s])
                  for c in range(CH)]
            del k
            return ht, hs

        def issue(k, s):
            b0 = base + k * CH
            pltpu.sync_copy(sents_hbm.at[pl.ds(b0, CH)], idxs[s])
            pltpu.sync_copy(mask_hbm.at[pl.ds(b0, CH)], msks[s])
            ht, hs = handles(k, s)
            for h in ht + hs:
                h.start()

        def consume(k, s):
            b0 = base + k * CH
            ht, hs = handles(k, s)

            def length(c):
                mrow = (msks[s][c, pl.ds(0, 16)] + msks[s][c, pl.ds(16, 16)]
                        + msks[s][c, pl.ds(32, 16)] + msks[s][c, pl.ds(48, 16)])
                return jnp.sum(mrow).astype(jnp.int32)

            def branch(rows_ref, sref, oref):
                def sent(c, _):
                    lenc = length(c)
                    scores_c(rows_ref, sref, c, lenc)
                    softmax_c(sref, c)
                    bag(rows_ref, sref, c, lenc, oref)
                    return 0

                lax.fori_loop(0, CH, sent, 0)

            for h in ht:
                h.wait()
            branch(rows_t[s], sc_t[s], out_t[s])
            for h in hs:
                h.wait()
            branch(rows_s[s], sc_s[s], out_s[s])
            pltpu.sync_copy(out_t[s], snt_t_hbm.at[pl.ds(b0, CH)])
            pltpu.sync_copy(out_s[s], snt_s_hbm.at[pl.ds(b0, CH)])

        issue(0, 0)

        def pair_body(i, _):
            k0 = i * 2
            issue(k0 + 1, 1)
            consume(k0, 0)
            issue(k0 + 2, 0)
            consume(k0 + 1, 1)
            return 0

        lax.fori_loop(0, NCHUNK // 2 - 1, pair_body, 0)
        issue(NCHUNK - 1, 1)
        consume(NCHUNK - 2, 0)
        consume(NCHUNK - 1, 1)

    return sc_kernel(sents, mask64, seeds_flat, att_sent, emb_t, emb_s)


def _tc_tail(snt_t, snt_s, sd_t, sd_s):
    BSZ = 512
    dn = (((1,), (1,)), ((), ()))

    def tc_body(snt_t_ref, snt_s_ref, sd_t_ref, sd_s_ref, out_ref):
        pro_t = lax.dot_general(snt_t_ref[...], sd_t_ref[...], dn,
                                preferred_element_type=jnp.float32)
        pro_s = lax.dot_general(snt_s_ref[...], sd_s_ref[...], dn,
                                preferred_element_type=jnp.float32)
        gt = jnp.concatenate(
            [jnp.max(pro_t[:, a * SPA:(a + 1) * SPA], axis=1, keepdims=True)
             for a in range(A)], axis=1)
        gs = jnp.concatenate(
            [jnp.max(pro_s[:, a * SPA:(a + 1) * SPA], axis=1, keepdims=True)
             for a in range(A)], axis=1)
        mt = jnp.max(gt, axis=1, keepdims=True)
        et = jnp.exp(gt - mt)
        pt = et / jnp.sum(et, axis=1, keepdims=True)
        reli = jnp.max(pt, axis=1, keepdims=True) - 1.0 / A
        dd = pt - gs
        out_ref[0, 0, 0] = jnp.sum((1.0 + ANPHA * jnp.abs(reli)) * dd * dd)

    return pl.pallas_call(
        tc_body,
        grid=(B // BSZ,),
        in_specs=[
            pl.BlockSpec((BSZ, D), lambda i: (i, 0)),
            pl.BlockSpec((BSZ, D), lambda i: (i, 0)),
            pl.BlockSpec((S_PAD, D), lambda i: (0, 0)),
            pl.BlockSpec((S_PAD, D), lambda i: (0, 0)),
        ],
        out_specs=pl.BlockSpec((1, 1, 1), lambda i: (i, 0, 0),
                               memory_space=pltpu.SMEM),
        out_shape=jax.ShapeDtypeStruct((B // BSZ, 1, 1), jnp.float32),
    )(snt_t, snt_s, sd_t, sd_s)


def kernel(sents, seeds, num_clusters, num_arr, mask, flag,
           emb_teacher, emb_student, att_sent):
    del num_clusters, num_arr
    seeds_flat = jnp.pad(seeds, ((0, S_PAD - S), (0, 0))).reshape(-1)
    snt_t, snt_s, sd_t, sd_s = _sc_embedding_stage(
        sents, mask, seeds_flat, att_sent, emb_teacher, emb_student)
    partials = _tc_tail(snt_t, snt_s, sd_t, sd_s)
    return jnp.sum(partials) / B * flag
